# baseline (speedup 1.0000x reference)
import functools

import jax
import jax.numpy as jnp
from jax import lax
from jax.experimental import pallas as pl
from jax.experimental.pallas import tpu as pltpu
from jax.experimental.pallas import tpu_sc as plsc

_LN = 128
_NU = 11
_ND = 8

_mesh = None


def _get_mesh():
    global _mesh
    if _mesh is None:
        _mesh = plsc.VectorSubcoreMesh(
            core_axis_name="c", subcore_axis_name="s")
    return _mesh


def _params():
    return pltpu.CompilerParams(
        needs_layout_passes=False, use_tc_tiling_on_sc=True)


def _make_pass_a(B, F, NC, NS, L):
    NW = NC * NS
    bw = B // NW
    nf = F // L
    ng = bw // L

    @functools.partial(
        pl.kernel,
        mesh=_get_mesh(),
        compiler_params=_params(),
        out_type=jax.ShapeDtypeStruct((B * F,), jnp.float32),
        scratch_types=[
            pltpu.VMEM((bw + L,), jnp.int32),
            pltpu.VMEM((bw + L,), jnp.int32),
            pltpu.VMEM((bw + L,), jnp.int32),
            pltpu.VMEM((_NU, F, _LN), jnp.float32),
            pltpu.VMEM((F + L,), jnp.float32),
            pltpu.VMEM((bw * F,), jnp.float32),
            pltpu.SemaphoreType.DMA,
        ],
    )
    def pass_a(us_hbm, f_hbm, m_hbm, utT_hbm, w_hbm, inter_hbm,
               us_v, f_v, m_v, uring, w_v, part_v, usem):
        wid = lax.axis_index("s") * NC + lax.axis_index("c")
        base = wid * bw

        pltpu.sync_copy(us_hbm.at[pl.ds(base, bw + L)], us_v)
        pltpu.sync_copy(f_hbm.at[pl.ds(base, bw + L)], f_v)
        pltpu.sync_copy(m_hbm.at[pl.ds(base, bw + L)], m_v)
        pltpu.sync_copy(w_hbm, w_v)

        wregs = [w_v[pl.ds(j * L, L)] for j in range(nf)]
        iot = lax.iota(jnp.int32, L)

        def ufetch(uidx_scalar, slot_scalar):
            start = pl.multiple_of((uidx_scalar // _LN) * _LN, _LN)
            pltpu.async_copy(
                utT_hbm.at[:, pl.ds(start, _LN)], uring.at[slot_scalar],
                usem)

        def udrain():
            pltpu.make_async_copy(
                utT_hbm.at[:, pl.ds(0, _LN)], uring.at[0], usem).wait()

        u0 = us_v[pl.ds(0, L)]
        f0 = f_v[pl.ds(0, L)]
        m0 = m_v[pl.ds(0, L)]
        for rr in range(_NU):
            @pl.when(f0[rr] == 1)
            def _():
                ufetch(u0[rr], m0[rr] % _NU)

        def group(g, carry):
            usg = us_v[pl.ds(g * L, L)]
            usgn = us_v[pl.ds(g * L + L, L)]
            fg = f_v[pl.ds(g * L, L)]
            fgn = f_v[pl.ds(g * L + L, L)]
            mg = m_v[pl.ds(g * L, L)]
            mgn = m_v[pl.ds(g * L + L, L)]
            for rr in range(L):
                @pl.when(fg[rr] == 1)
                def _():
                    udrain()
                ku = jnp.full((L,), mg[rr] % _NU, jnp.int32)
                ul = jnp.full((L,), usg[rr] % _LN, jnp.int32)
                qbase = (g * L + rr) * F
                for j in range(nf):
                    u = plsc.load_gather(uring, [ku, iot + j * L, ul])
                    part_v[pl.ds(qbase + j * L, L)] = u * wregs[j]
                if rr + _NU < L:
                    cond = fg[rr + _NU] == 1
                    col8 = usg[rr + _NU]
                    slot8 = mg[rr + _NU] % _NU
                else:
                    cond = jnp.logical_and(
                        fgn[rr + _NU - L] == 1, g < ng - 1)
                    col8 = usgn[rr + _NU - L]
                    slot8 = mgn[rr + _NU - L] % _NU

                @pl.when(cond)
                def _():
                    ufetch(col8, slot8)
            return carry

        lax.fori_loop(0, ng, group, 0)
        pltpu.sync_copy(part_v, inter_hbm.at[pl.ds(base * F, bw * F)])

    return pass_a


def _make_pass_b(B, F, NC, NS, L):
    NW = NC * NS
    bw = B // NW
    nf = F // L
    ng = bw // L

    @functools.partial(
        pl.kernel,
        mesh=_get_mesh(),
        compiler_params=_params(),
        out_type=jax.ShapeDtypeStruct((B,), jnp.float32),
        scratch_types=[
            pltpu.VMEM((bw + L,), jnp.int32),
            pltpu.VMEM((bw + L,), jnp.int32),
            pltpu.VMEM((bw + L,), jnp.int32),
            pltpu.VMEM((bw + L,), jnp.int32),
            pltpu.VMEM((_NU, F, _LN), jnp.float32),
            pltpu.VMEM((_ND, F), jnp.float32),
            pltpu.VMEM((F + L,), jnp.float32),
            pltpu.VMEM((bw,), jnp.float32),
            pltpu.SemaphoreType.DMA,
            pltpu.SemaphoreType.DMA,
        ],
    )
    def pass_b(is_hbm, f_hbm, m_hbm, pu_hbm, itT_hbm, w_hbm, inter_hbm,
               out_hbm, is_v, f_v, m_v, pu_v, iring, pring, w_v, out_v,
               isem, psem):
        wid = lax.axis_index("s") * NC + lax.axis_index("c")
        base = wid * bw

        pltpu.sync_copy(is_hbm.at[pl.ds(base, bw + L)], is_v)
        pltpu.sync_copy(f_hbm.at[pl.ds(base, bw + L)], f_v)
        pltpu.sync_copy(m_hbm.at[pl.ds(base, bw + L)], m_v)
        pltpu.sync_copy(pu_hbm.at[pl.ds(base, bw + L)], pu_v)
        pltpu.sync_copy(w_hbm, w_v)

        b_s = w_v[pl.ds(F, L)][0]
        iot = lax.iota(jnp.int32, L)

        def ifetch(idx_scalar, slot_scalar):
            start = pl.multiple_of((idx_scalar // _LN) * _LN, _LN)
            pltpu.async_copy(
                itT_hbm.at[:, pl.ds(start, _LN)], iring.at[slot_scalar],
                isem)

        def idrain():
            pltpu.make_async_copy(
                itT_hbm.at[:, pl.ds(0, _LN)], iring.at[0], isem).wait()

        def pread(pos_scalar, k):
            pltpu.async_copy(
                inter_hbm.at[pl.ds(pos_scalar * F, F)], pring.at[k], psem)

        def pwait():
            pltpu.make_async_copy(
                inter_hbm.at[pl.ds(0, F)], pring.at[0], psem).wait()

        i0 = is_v[pl.ds(0, L)]
        f0 = f_v[pl.ds(0, L)]
        m0 = m_v[pl.ds(0, L)]
        pu0 = pu_v[pl.ds(0, L)]
        for rr in range(_NU):
            @pl.when(f0[rr] == 1)
            def _():
                ifetch(i0[rr], m0[rr] % _NU)
        for rr in range(_ND):
            pread(pu0[rr], rr)

        def group(g, carry):
            isg = is_v[pl.ds(g * L, L)]
            isgn = is_v[pl.ds(g * L + L, L)]
            fg = f_v[pl.ds(g * L, L)]
            fgn = f_v[pl.ds(g * L + L, L)]
            mg = m_v[pl.ds(g * L, L)]
            mgn = m_v[pl.ds(g * L + L, L)]
            pug = pu_v[pl.ds(g * L, L)]
            pugn = pu_v[pl.ds(g * L + L, L)]
            yv = jnp.zeros((L,), jnp.float32)
            for rr in range(L):
                k = rr % _ND

                @pl.when(fg[rr] == 1)
                def _():
                    idrain()
                pwait()
                ki = jnp.full((L,), mg[rr] % _NU, jnp.int32)
                il = jnp.full((L,), isg[rr] % _LN, jnp.int32)
                acc = None
                for j in range(nf):
                    pw = pring[k, pl.ds(j * L, L)]
                    it = plsc.load_gather(iring, [ki, iot + j * L, il])
                    p = pw * it
                    acc = p if acc is None else acc + p
                if rr + _ND < L:
                    pread(pug[rr + _ND], k)
                else:
                    pread(pugn[rr + _ND - L], k)
                if rr + _NU < L:
                    cond = fg[rr + _NU] == 1
                    col8 = isg[rr + _NU]
                    slot8 = mg[rr + _NU] % _NU
                else:
                    cond = jnp.logical_and(
                        fgn[rr + _NU - L] == 1, g < ng - 1)
                    col8 = isgn[rr + _NU - L]
                    slot8 = mgn[rr + _NU - L] % _NU

                @pl.when(cond)
                def _():
                    ifetch(col8, slot8)

                s = jnp.sum(acc)
                yv = jnp.where(iot == rr, s, yv)
            z = yv + b_s
            out_v[pl.ds(g * L, L)] = 1.0 / (1.0 + jnp.exp(-z))
            return carry

        lax.fori_loop(0, ng, group, 0)
        for _ in range(_ND):
            pwait()
        pltpu.sync_copy(out_v, out_hbm.at[pl.ds(base, bw)])

    return pass_b


def _side_arrays(srt, perm, B, bw, L):
    arange = jnp.arange(B, dtype=jnp.int32)
    inv = jnp.zeros((B,), jnp.int32).at[perm].set(arange)
    col = srt // _LN
    f = ((col != jnp.roll(col, 1)) | (arange % bw == 0)).astype(jnp.int32)
    m = jnp.cumsum(f).astype(jnp.int32) - 1
    zpad = jnp.zeros((L,), jnp.int32)
    pad = lambda a: jnp.concatenate([a, zpad])
    return inv, pad(srt), pad(f), pad(m)


def kernel(user_indices, item_indices, user_table, item_table, W, b):
    B = user_indices.shape[0]
    F = user_table.shape[1]
    info = plsc.get_sparse_core_info()
    NC, NS, L = info.num_cores, info.num_subcores, info.num_lanes
    NW = NC * NS
    bw = B // NW

    uidx = user_indices.astype(jnp.int32)
    iidx = item_indices.astype(jnp.int32)

    arange2 = jnp.broadcast_to(jnp.arange(B, dtype=jnp.int32), (2, B))
    keys = jnp.stack([uidx, iidx])
    skeys, sperm = lax.sort_key_val(keys, arange2, dimension=1)
    up, ip = sperm[0], sperm[1]
    uinv, us_p, fu_p, mu_p = _side_arrays(skeys[0], up, B, bw, L)
    iinv, is_p, fi_p, mi_p = _side_arrays(skeys[1], ip, B, bw, L)
    pu = uinv[ip]
    pu_p = jnp.concatenate([pu, jnp.zeros((L,), jnp.int32)])

    wpad = jnp.concatenate([
        W.reshape(F).astype(jnp.float32),
        b.reshape(1).astype(jnp.float32),
        jnp.zeros((L - 1,), jnp.float32),
    ])

    pass_a = _make_pass_a(B, F, NC, NS, L)
    pass_b = _make_pass_b(B, F, NC, NS, L)
    inter = pass_a(us_p, fu_p, mu_p, user_table.T, wpad)
    y_sorted = pass_b(is_p, fi_p, mi_p, pu_p, item_table.T, wpad, inter)
    return y_sorted[iinv].reshape(B, 1)


# speedup vs baseline: 1.1275x; 1.1275x over previous
import functools

import jax
import jax.numpy as jnp
from jax import lax
from jax.experimental import pallas as pl
from jax.experimental.pallas import tpu as pltpu
from jax.experimental.pallas import tpu_sc as plsc

_LN = 128
_NU = 11
_ND = 8

_mesh = None


def _get_mesh():
    global _mesh
    if _mesh is None:
        _mesh = plsc.VectorSubcoreMesh(
            core_axis_name="c", subcore_axis_name="s")
    return _mesh


def _params():
    return pltpu.CompilerParams(
        needs_layout_passes=False, use_tc_tiling_on_sc=True)


def _make_pass_a(B, F, NC, NS, L):
    NW = NC * NS
    bw = B // NW
    nf = F // L
    ng = bw // L

    @functools.partial(
        pl.kernel,
        mesh=_get_mesh(),
        compiler_params=_params(),
        out_type=jax.ShapeDtypeStruct((B * F,), jnp.float32),
        scratch_types=[
            pltpu.VMEM((bw + L,), jnp.int32),
            pltpu.VMEM((bw + L,), jnp.int32),
            pltpu.VMEM((bw + L,), jnp.int32),
            pltpu.VMEM((_NU, F, _LN), jnp.float32),
            pltpu.VMEM((F + L,), jnp.float32),
            pltpu.VMEM((bw * F,), jnp.float32),
            pltpu.SemaphoreType.DMA,
        ],
    )
    def pass_a(us_hbm, f_hbm, m_hbm, utT_hbm, w_hbm, inter_hbm,
               us_v, f_v, m_v, uring, w_v, part_v, usem):
        wid = lax.axis_index("s") * NC + lax.axis_index("c")
        base = wid * bw

        pltpu.sync_copy(us_hbm.at[pl.ds(base, bw + L)], us_v)
        pltpu.sync_copy(f_hbm.at[pl.ds(base, bw + L)], f_v)
        pltpu.sync_copy(m_hbm.at[pl.ds(base, bw + L)], m_v)
        pltpu.sync_copy(w_hbm, w_v)

        wregs = [w_v[pl.ds(j * L, L)] for j in range(nf)]
        iot = lax.iota(jnp.int32, L)

        def ufetch(uidx_scalar, slot_scalar):
            start = pl.multiple_of((uidx_scalar // _LN) * _LN, _LN)
            pltpu.async_copy(
                utT_hbm.at[:, pl.ds(start, _LN)], uring.at[slot_scalar],
                usem)

        def udrain():
            pltpu.make_async_copy(
                utT_hbm.at[:, pl.ds(0, _LN)], uring.at[0], usem).wait()

        u0 = us_v[pl.ds(0, L)]
        f0 = f_v[pl.ds(0, L)]
        m0 = m_v[pl.ds(0, L)]
        for rr in range(_NU):
            @pl.when(f0[rr] == 1)
            def _():
                ufetch(u0[rr], m0[rr] % _NU)

        def group(g, carry):
            usg = us_v[pl.ds(g * L, L)]
            usgn = us_v[pl.ds(g * L + L, L)]
            fg = f_v[pl.ds(g * L, L)]
            fgn = f_v[pl.ds(g * L + L, L)]
            mg = m_v[pl.ds(g * L, L)]
            mgn = m_v[pl.ds(g * L + L, L)]
            for rr in range(L):
                @pl.when(fg[rr] == 1)
                def _():
                    udrain()
                ku = jnp.full((L,), mg[rr] % _NU, jnp.int32)
                ul = jnp.full((L,), usg[rr] % _LN, jnp.int32)
                qbase = (g * L + rr) * F
                for j in range(nf):
                    u = plsc.load_gather(uring, [ku, iot + j * L, ul])
                    part_v[pl.ds(qbase + j * L, L)] = u * wregs[j]
                if rr + _NU < L:
                    cond = fg[rr + _NU] == 1
                    col8 = usg[rr + _NU]
                    slot8 = mg[rr + _NU] % _NU
                else:
                    cond = jnp.logical_and(
                        fgn[rr + _NU - L] == 1, g < ng - 1)
                    col8 = usgn[rr + _NU - L]
                    slot8 = mgn[rr + _NU - L] % _NU

                @pl.when(cond)
                def _():
                    ufetch(col8, slot8)
            return carry

        lax.fori_loop(0, ng, group, 0)
        pltpu.sync_copy(part_v, inter_hbm.at[pl.ds(base * F, bw * F)])

    return pass_a


def _make_pass_b(B, F, NC, NS, L):
    NW = NC * NS
    bw = B // NW
    nf = F // L
    ng = bw // L

    @functools.partial(
        pl.kernel,
        mesh=_get_mesh(),
        compiler_params=_params(),
        out_type=jax.ShapeDtypeStruct((B,), jnp.float32),
        scratch_types=[
            pltpu.VMEM((bw + L,), jnp.int32),
            pltpu.VMEM((bw + L,), jnp.int32),
            pltpu.VMEM((bw + L,), jnp.int32),
            pltpu.VMEM((bw + L,), jnp.int32),
            pltpu.VMEM((_NU, F, _LN), jnp.float32),
            pltpu.VMEM((_ND, F), jnp.float32),
            pltpu.VMEM((F + L,), jnp.float32),
            pltpu.VMEM((bw,), jnp.float32),
            pltpu.SemaphoreType.DMA,
            pltpu.SemaphoreType.DMA,
        ],
    )
    def pass_b(is_hbm, f_hbm, m_hbm, pu_hbm, itT_hbm, w_hbm, inter_hbm,
               out_hbm, is_v, f_v, m_v, pu_v, iring, pring, w_v, out_v,
               isem, psem):
        wid = lax.axis_index("s") * NC + lax.axis_index("c")
        base = wid * bw

        pltpu.sync_copy(is_hbm.at[pl.ds(base, bw + L)], is_v)
        pltpu.sync_copy(f_hbm.at[pl.ds(base, bw + L)], f_v)
        pltpu.sync_copy(m_hbm.at[pl.ds(base, bw + L)], m_v)
        pltpu.sync_copy(pu_hbm.at[pl.ds(base, bw + L)], pu_v)
        pltpu.sync_copy(w_hbm, w_v)

        b_s = w_v[pl.ds(F, L)][0]
        iot = lax.iota(jnp.int32, L)

        def ifetch(idx_scalar, slot_scalar):
            start = pl.multiple_of((idx_scalar // _LN) * _LN, _LN)
            pltpu.async_copy(
                itT_hbm.at[:, pl.ds(start, _LN)], iring.at[slot_scalar],
                isem)

        def idrain():
            pltpu.make_async_copy(
                itT_hbm.at[:, pl.ds(0, _LN)], iring.at[0], isem).wait()

        def pread(pos_scalar, k):
            pltpu.async_copy(
                inter_hbm.at[pl.ds(pos_scalar * F, F)], pring.at[k], psem)

        def pwait():
            pltpu.make_async_copy(
                inter_hbm.at[pl.ds(0, F)], pring.at[0], psem).wait()

        i0 = is_v[pl.ds(0, L)]
        f0 = f_v[pl.ds(0, L)]
        m0 = m_v[pl.ds(0, L)]
        pu0 = pu_v[pl.ds(0, L)]
        for rr in range(_NU):
            @pl.when(f0[rr] == 1)
            def _():
                ifetch(i0[rr], m0[rr] % _NU)
        for rr in range(_ND):
            pread(pu0[rr], rr)

        def group(g, carry):
            isg = is_v[pl.ds(g * L, L)]
            isgn = is_v[pl.ds(g * L + L, L)]
            fg = f_v[pl.ds(g * L, L)]
            fgn = f_v[pl.ds(g * L + L, L)]
            mg = m_v[pl.ds(g * L, L)]
            mgn = m_v[pl.ds(g * L + L, L)]
            pug = pu_v[pl.ds(g * L, L)]
            pugn = pu_v[pl.ds(g * L + L, L)]
            yv = jnp.zeros((L,), jnp.float32)
            for rr in range(L):
                k = rr % _ND

                @pl.when(fg[rr] == 1)
                def _():
                    idrain()
                pwait()
                ki = jnp.full((L,), mg[rr] % _NU, jnp.int32)
                il = jnp.full((L,), isg[rr] % _LN, jnp.int32)
                acc = None
                for j in range(nf):
                    pw = pring[k, pl.ds(j * L, L)]
                    it = plsc.load_gather(iring, [ki, iot + j * L, il])
                    p = pw * it
                    acc = p if acc is None else acc + p
                if rr + _ND < L:
                    pread(pug[rr + _ND], k)
                else:
                    pread(pugn[rr + _ND - L], k)
                if rr + _NU < L:
                    cond = fg[rr + _NU] == 1
                    col8 = isg[rr + _NU]
                    slot8 = mg[rr + _NU] % _NU
                else:
                    cond = jnp.logical_and(
                        fgn[rr + _NU - L] == 1, g < ng - 1)
                    col8 = isgn[rr + _NU - L]
                    slot8 = mgn[rr + _NU - L] % _NU

                @pl.when(cond)
                def _():
                    ifetch(col8, slot8)

                s = jnp.sum(acc)
                yv = jnp.where(iot == rr, s, yv)
            z = yv + b_s
            out_v[pl.ds(g * L, L)] = 1.0 / (1.0 + jnp.exp(-z))
            return carry

        lax.fori_loop(0, ng, group, 0)
        for _ in range(_ND):
            pwait()
        pltpu.sync_copy(out_v, out_hbm.at[pl.ds(base, bw)])

    return pass_b


def _sorted_side(idx, B, bw, L):
    arange = jnp.arange(B, dtype=jnp.int32)
    perm = jnp.argsort(idx)
    srt = idx[perm]
    inv = jnp.zeros((B,), jnp.int32).at[perm].set(arange)
    col = srt // _LN
    f = ((col != jnp.roll(col, 1)) | (arange % bw == 0)).astype(jnp.int32)
    m = jnp.cumsum(f).astype(jnp.int32) - 1
    zpad = jnp.zeros((L,), jnp.int32)
    pad = lambda a: jnp.concatenate([a, zpad])
    return perm, inv, pad(srt), pad(f), pad(m)


def kernel(user_indices, item_indices, user_table, item_table, W, b):
    B = user_indices.shape[0]
    F = user_table.shape[1]
    info = plsc.get_sparse_core_info()
    NC, NS, L = info.num_cores, info.num_subcores, info.num_lanes
    NW = NC * NS
    bw = B // NW

    uidx = user_indices.astype(jnp.int32)
    iidx = item_indices.astype(jnp.int32)

    up, uinv, us_p, fu_p, mu_p = _sorted_side(uidx, B, bw, L)
    ip, iinv, is_p, fi_p, mi_p = _sorted_side(iidx, B, bw, L)
    pu = uinv[ip]
    pu_p = jnp.concatenate([pu, jnp.zeros((L,), jnp.int32)])

    wpad = jnp.concatenate([
        W.reshape(F).astype(jnp.float32),
        b.reshape(1).astype(jnp.float32),
        jnp.zeros((L - 1,), jnp.float32),
    ])

    pass_a = _make_pass_a(B, F, NC, NS, L)
    pass_b = _make_pass_b(B, F, NC, NS, L)
    inter = pass_a(us_p, fu_p, mu_p, user_table.T, wpad)
    y_sorted = pass_b(is_p, fi_p, mi_p, pu_p, item_table.T, wpad, inter)
    return y_sorted[iinv].reshape(B, 1)
